# baseline (speedup 1.0000x reference)
import functools

import jax
import jax.numpy as jnp
from jax import lax
from jax.experimental import pallas as pl
from jax.experimental.pallas import tpu as pltpu
from jax.experimental.pallas import tpu_sc as plsc


def kernel(unary, binary, index1, index2):
    N, D = unary.shape
    E, DB = binary.shape
    DO = 2 * D + DB
    L = 128

    info = plsc.get_sparse_core_info()
    NW = info.num_cores * info.num_subcores
    NC = info.num_cores

    assert E % L == 0
    n_groups = E // L
    gpw = n_groups // NW
    rem = n_groups - gpw * NW
    max_g = gpw + (1 if rem else 0)
    stage = max_g * L

    index1 = index1.astype(jnp.int32)
    index2 = index2.astype(jnp.int32)
    binary_t = binary.T

    mesh = plsc.VectorSubcoreMesh(core_axis_name="c", subcore_axis_name="s")

    @functools.partial(
        pl.kernel,
        mesh=mesh,
        out_type=jax.ShapeDtypeStruct((DO, E), jnp.float32),
        scratch_types=[
            pltpu.VMEM((stage,), jnp.int32),
            pltpu.VMEM((stage,), jnp.int32),
            pltpu.VMEM((L, D), jnp.float32),
            pltpu.VMEM((L, D), jnp.float32),
            pltpu.VMEM((L, D), jnp.float32),
            pltpu.VMEM((L, D), jnp.float32),
            pltpu.VMEM((DO, L), jnp.float32),
            pltpu.SemaphoreType.DMA,
            pltpu.SemaphoreType.DMA,
            pltpu.SemaphoreType.DMA,
            pltpu.SemaphoreType.DMA,
        ],
        compiler_params=pltpu.CompilerParams(
            use_tc_tiling_on_sc=True, needs_layout_passes=False),
    )
    def join(unary_hbm, bin_t_hbm, idx1_hbm, idx2_hbm, out_t_hbm,
             idx1_v, idx2_v, r1a, r2a, r1b, r2b, outb,
             gsem_a, gsem_b, bsem, wsem):
        w = lax.axis_index("s") * NC + lax.axis_index("c")
        cnt = gpw + jnp.where(w < rem, 1, 0)
        base_g = gpw * w + jnp.minimum(w, rem)
        base_e = base_g * L

        pltpu.sync_copy(idx1_hbm.at[pl.ds(base_e, stage)], idx1_v)
        pltpu.sync_copy(idx2_hbm.at[pl.ds(base_e, stage)], idx2_v)

        slots = ((r1a, r2a, gsem_a), (r1b, r2b, gsem_b))

        def issue_gathers(i, s):
            r1, r2, gs = slots[s]
            pltpu.async_copy(unary_hbm.at[idx1_v.at[pl.ds(i * L, L)]], r1, gs)
            pltpu.async_copy(unary_hbm.at[idx2_v.at[pl.ds(i * L, L)]], r2, gs)

        def wait_gathers(s):
            r1, r2, gs = slots[s]
            pltpu.make_async_copy(unary_hbm.at[pl.ds(0, L)], r1, gs).wait()
            pltpu.make_async_copy(unary_hbm.at[pl.ds(0, L)], r2, gs).wait()

        def wait_write():
            pltpu.make_async_copy(
                outb, out_t_hbm.at[:, pl.ds(0, L)], wsem).wait()

        iota = lax.iota(jnp.int32, 16)
        segs = [iota + 16 * s for s in range(8)]

        def transpose_into(rref, row0):
            def frow(f, carry):
                fv = jnp.full((16,), f, jnp.int32)
                for s in range(8):
                    v = plsc.load_gather(rref, [segs[s], fv])
                    outb[row0 + f, pl.ds(16 * s, 16)] = v
                return carry
            lax.fori_loop(0, D, frow, 0, unroll=2)

        issue_gathers(0, 0)
        issue_gathers(1, 1)

        def pair(g, carry):
            for s in range(2):
                i = 2 * g + s

                @pl.when(i < cnt)
                def _():
                    col = (base_g + i) * L
                    wait_gathers(s)

                    @pl.when(i >= 1)
                    def _():
                        wait_write()

                    pltpu.async_copy(
                        bin_t_hbm.at[:, pl.ds(col, L)],
                        outb.at[pl.ds(2 * D, DB), :], bsem)
                    r1, r2, _gs = slots[s]
                    transpose_into(r1, 0)
                    transpose_into(r2, D)
                    pltpu.make_async_copy(
                        bin_t_hbm.at[:, pl.ds(0, L)],
                        outb.at[pl.ds(2 * D, DB), :], bsem).wait()
                    pltpu.async_copy(
                        outb, out_t_hbm.at[:, pl.ds(col, L)], wsem)

                    @pl.when(i + 2 < cnt)
                    def _():
                        issue_gathers(i + 2, s)

            return carry

        lax.fori_loop(0, (max_g + 1) // 2, pair, 0)
        wait_write()

    out_t = join(unary, binary_t, index1, index2)
    return out_t.T


# speedup vs baseline: 1.7962x; 1.7962x over previous
import functools

import jax
import jax.numpy as jnp
from jax import lax
from jax.experimental import pallas as pl
from jax.experimental.pallas import tpu as pltpu
from jax.experimental.pallas import tpu_sc as plsc


def kernel(unary, binary, index1, index2):
    N, D = unary.shape
    E, DB = binary.shape
    DO = 2 * D + DB
    L = 128

    info = plsc.get_sparse_core_info()
    NW = info.num_cores * info.num_subcores
    NC = info.num_cores

    assert E % L == 0
    n_groups = E // L
    gpw = n_groups // NW
    rem = n_groups - gpw * NW
    max_g = gpw + (1 if rem else 0)
    stage = max_g * L

    index1 = index1.astype(jnp.int32)
    index2 = index2.astype(jnp.int32)
    binary_t = binary.T

    mesh = plsc.VectorSubcoreMesh(core_axis_name="c", subcore_axis_name="s")

    @functools.partial(
        pl.kernel,
        mesh=mesh,
        out_type=jax.ShapeDtypeStruct((DO, E), jnp.float32),
        scratch_types=[
            pltpu.VMEM((stage,), jnp.int32),
            pltpu.VMEM((stage,), jnp.int32),
            pltpu.VMEM((L, D), jnp.float32),
            pltpu.VMEM((L, D), jnp.float32),
            pltpu.VMEM((L, D), jnp.float32),
            pltpu.VMEM((L, D), jnp.float32),
            pltpu.VMEM((DO, L), jnp.float32),
            pltpu.SemaphoreType.DMA,
            pltpu.SemaphoreType.DMA,
            pltpu.SemaphoreType.DMA,
            pltpu.SemaphoreType.DMA,
        ],
        compiler_params=pltpu.CompilerParams(
            use_tc_tiling_on_sc=True, needs_layout_passes=False),
    )
    def join(unary_hbm, bin_t_hbm, idx1_hbm, idx2_hbm, out_t_hbm,
             idx1_v, idx2_v, r1a, r2a, r1b, r2b, outb,
             gsem_a, gsem_b, bsem, wsem):
        w = lax.axis_index("s") * NC + lax.axis_index("c")
        cnt = gpw + jnp.where(w < rem, 1, 0)
        base_g = gpw * w + jnp.minimum(w, rem)
        base_e = base_g * L

        pltpu.sync_copy(idx1_hbm.at[pl.ds(base_e, stage)], idx1_v)
        pltpu.sync_copy(idx2_hbm.at[pl.ds(base_e, stage)], idx2_v)

        slots = ((r1a, r2a, gsem_a), (r1b, r2b, gsem_b))

        def issue_gathers(i, s):
            r1, r2, gs = slots[s]
            pltpu.async_copy(unary_hbm.at[idx1_v.at[pl.ds(i * L, L)]], r1, gs)
            pltpu.async_copy(unary_hbm.at[idx2_v.at[pl.ds(i * L, L)]], r2, gs)

        def wait_gathers(s):
            r1, r2, gs = slots[s]
            pltpu.make_async_copy(unary_hbm.at[pl.ds(0, L)], r1, gs).wait()
            pltpu.make_async_copy(unary_hbm.at[pl.ds(0, L)], r2, gs).wait()

        def wait_write():
            pltpu.make_async_copy(
                outb, out_t_hbm.at[:, pl.ds(0, L)], wsem).wait()

        iota = lax.iota(jnp.int32, 16)
        segs = [iota + 16 * s for s in range(8)]

        def transpose_into(rref, row0):
            @plsc.parallel_loop(0, D, unroll=4)
            def frow(f):
                fv = jnp.full((16,), f, jnp.int32)
                for s in range(8):
                    v = plsc.load_gather(rref, [segs[s], fv])
                    outb[row0 + f, pl.ds(16 * s, 16)] = v

        issue_gathers(0, 0)
        issue_gathers(1, 1)

        def pair(g, carry):
            for s in range(2):
                i = 2 * g + s

                @pl.when(i < cnt)
                def _():
                    col = (base_g + i) * L
                    wait_gathers(s)

                    @pl.when(i >= 1)
                    def _():
                        wait_write()

                    pltpu.async_copy(
                        bin_t_hbm.at[:, pl.ds(col, L)],
                        outb.at[pl.ds(2 * D, DB), :], bsem)
                    r1, r2, _gs = slots[s]
                    transpose_into(r1, 0)
                    transpose_into(r2, D)
                    pltpu.make_async_copy(
                        bin_t_hbm.at[:, pl.ds(0, L)],
                        outb.at[pl.ds(2 * D, DB), :], bsem).wait()
                    pltpu.async_copy(
                        outb, out_t_hbm.at[:, pl.ds(col, L)], wsem)

                    @pl.when(i + 2 < cnt)
                    def _():
                        issue_gathers(i + 2, s)

            return carry

        lax.fori_loop(0, (max_g + 1) // 2, pair, 0)
        wait_write()

    out_t = join(unary, binary_t, index1, index2)
    return out_t.T
